# baseline (speedup 1.0000x reference)
import functools

import jax
import jax.numpy as jnp
from jax import lax
from jax.experimental import pallas as pl
from jax.experimental.pallas import tpu as pltpu
from jax.experimental.pallas import tpu_sc as plsc

N = 4096
K = 512
J = 8192
NB = 512
JC = 2048
NI = N // NB
NJC = J // JC
BETA = 0.25
WEIGHT_DECAY = 0.01


def _unfold(z):
    x = z[:, :, None, :, :]
    b, c = x.shape[0], x.shape[1]
    x = jax.image.resize(x, (b, c, 2, 2, 2), method='trilinear')
    b_, c_, d_, h_, w_ = x.shape
    x = x.reshape(b_, c_, d_ // 2, 2, h_ // 2, 2, w_ // 2, 2)
    x = jnp.transpose(x, (0, 1, 3, 5, 7, 2, 4, 6))
    x = x.reshape(b_, c_ * 8, d_ // 2, h_ // 2, w_ // 2)
    x = jnp.squeeze(x, axis=2)
    x = jnp.transpose(x, (0, 2, 3, 1))
    return x.reshape(-1, K)


def _dist_argmin_kernel(s1_ref, s2_ref, zf_ref, w_ref, idx_ref):
    a = zf_ref[...]
    s1 = s1_ref[...]

    def body(jc, carry):
        best, bidx = carry
        wc = w_ref[pl.ds(jc * JC, JC), :]
        mm = lax.dot_general(a, wc, (((1,), (1,)), ((), ())),
                             preferred_element_type=jnp.float32)
        d = (s1 + s2_ref[:, pl.ds(jc * JC, JC)]) - 2.0 * mm
        lmin = jnp.min(d, axis=1, keepdims=True)
        col = lax.broadcasted_iota(jnp.int32, (NB, JC), 1)
        larg = jnp.min(jnp.where(d == lmin, col, J), axis=1, keepdims=True)
        larg = larg + jc * JC
        take = lmin < best
        return (jnp.where(take, lmin, best), jnp.where(take, larg, bidx))

    best0 = jnp.full((NB, 1), jnp.inf, jnp.float32)
    bidx0 = jnp.zeros((NB, 1), jnp.int32)
    _, bidx = lax.fori_loop(0, NJC, body, (best0, bidx0))
    idx_ref[0, :, :] = bidx


def _dist_argmin(zfb, wb, s1, s2):
    idx = pl.pallas_call(
        _dist_argmin_kernel,
        grid=(NI,),
        in_specs=[
            pl.BlockSpec((NB, 1), lambda i: (i, 0)),
            pl.BlockSpec((1, J), lambda i: (0, 0)),
            pl.BlockSpec((NB, K), lambda i: (i, 0)),
            pl.BlockSpec((J, K), lambda i: (0, 0)),
        ],
        out_specs=pl.BlockSpec((1, NB, 1), lambda i: (i, 0, 0)),
        out_shape=jax.ShapeDtypeStruct((NI, NB, 1), jnp.int32),
        compiler_params=pltpu.CompilerParams(
            dimension_semantics=("parallel",)),
    )(s1, s2, zfb, wb)
    return idx.reshape(N)


def _sc_gather(w, idx):
    info = plsc.get_sparse_core_info()
    nw = info.num_cores * info.num_subcores
    b_per_w = N // nw
    mesh = plsc.VectorSubcoreMesh(core_axis_name="c", subcore_axis_name="s")

    @functools.partial(
        pl.kernel, mesh=mesh,
        out_type=jax.ShapeDtypeStruct((N, K), jnp.float32),
        scratch_types=[
            pltpu.VMEM((b_per_w,), jnp.int32),
            pltpu.VMEM((b_per_w, K), jnp.float32),
            pltpu.SemaphoreType.DMA,
        ],
    )
    def gather_kernel(table_hbm, idx_hbm, out_hbm, idx_v, rows_v, sem):
        wid = lax.axis_index("s") * info.num_cores + lax.axis_index("c")
        base = wid * b_per_w
        pltpu.sync_copy(idx_hbm.at[pl.ds(base, b_per_w)], idx_v)
        pltpu.async_copy(table_hbm.at[idx_v], rows_v, sem).wait()
        pltpu.sync_copy(rows_v, out_hbm.at[pl.ds(base, b_per_w)])

    return gather_kernel(w, idx)


_RB = 512
_RG = N // _RB
_WB = J // _RG


def _loss_out_kernel(z_ref, zq_ref, w_ref, out_ref, sums_ref, colsum_ref):
    z = z_ref[...]
    zq = zq_ref[...]
    diff = zq - z
    out_ref[...] = z + diff

    p = jnp.stack([
        jnp.sum(diff * diff),
        jnp.sum(zq),
        jnp.sum(z),
        jnp.sum(zq * z),
        jnp.sum(zq * zq),
        jnp.sum(z * z),
        jnp.float32(0.0), jnp.float32(0.0),
    ])
    sums_ref[0, 0, :] = p

    wblk = w_ref[...].astype(jnp.float32)
    colsum_ref[0, :, :] = jnp.sum(jnp.abs(wblk), axis=0, keepdims=True)


def _loss_out(z_flat, zq, wb):
    out, sums, colsums = pl.pallas_call(
        _loss_out_kernel,
        grid=(_RG,),
        in_specs=[
            pl.BlockSpec((_RB, K), lambda g: (g, 0)),
            pl.BlockSpec((_RB, K), lambda g: (g, 0)),
            pl.BlockSpec((_WB, K), lambda g: (g, 0)),
        ],
        out_specs=[
            pl.BlockSpec((_RB, K), lambda g: (g, 0)),
            pl.BlockSpec((1, 1, 8), lambda g: (g, 0, 0)),
            pl.BlockSpec((1, 1, K), lambda g: (g, 0, 0)),
        ],
        out_shape=[
            jax.ShapeDtypeStruct((N, K), jnp.float32),
            jax.ShapeDtypeStruct((_RG, 1, 8), jnp.float32),
            jax.ShapeDtypeStruct((_RG, 1, K), jnp.float32),
        ],
        compiler_params=pltpu.CompilerParams(
            dimension_semantics=("parallel",)),
    )(z_flat, zq, wb)
    reg = WEIGHT_DECAY * jnp.max(jnp.sum(colsums.reshape(_RG, K), axis=0))
    return out, jnp.sum(sums.reshape(_RG, 8), axis=0), reg


def kernel(z, embedding_weight):
    w = embedding_weight
    zf = _unfold(z)
    s1 = jnp.sum(zf ** 2, axis=1, keepdims=True)
    s2 = jnp.sum(w ** 2, axis=1).reshape(1, J)
    zfb = zf.astype(jnp.bfloat16)
    wb = w.astype(jnp.bfloat16)

    idx = jnp.zeros((N,), jnp.int32)
    zq = _sc_gather(w, idx)

    z_flat = z.reshape(N, K)
    out_flat, sums, reg = _loss_out(z_flat, zq, wb)

    n_tot = jnp.float32(N * K)
    s_d2, s_q, s_z, s_qz, s_q2, s_z2 = [sums[k] for k in range(6)]
    sxy = s_qz - s_q * s_z / n_tot
    sxx = s_q2 - s_q * s_q / n_tot
    syy = s_z2 - s_z * s_z / n_tot
    cost = sxy / (jnp.sqrt(sxx) * jnp.sqrt(syy))
    pearson = 0.5 + 0.5 * cost
    m = s_d2 / n_tot
    loss = BETA * m + m + pearson + reg

    out = jnp.transpose(out_flat.reshape(z.shape), (0, 3, 1, 2))
    return out, loss, idx


# speedup vs baseline: 1.0233x; 1.0233x over previous
import functools

import jax
import jax.numpy as jnp
from jax import lax
from jax.experimental import pallas as pl
from jax.experimental.pallas import tpu as pltpu
from jax.experimental.pallas import tpu_sc as plsc

N = 4096
K = 512
J = 8192
NB = 512
JC = 2048
NI = N // NB
NJC = J // JC
BETA = 0.25
WEIGHT_DECAY = 0.01


def _unfold(z):
    x = z[:, :, None, :, :]
    b, c = x.shape[0], x.shape[1]
    x = jax.image.resize(x, (b, c, 2, 2, 2), method='trilinear')
    b_, c_, d_, h_, w_ = x.shape
    x = x.reshape(b_, c_, d_ // 2, 2, h_ // 2, 2, w_ // 2, 2)
    x = jnp.transpose(x, (0, 1, 3, 5, 7, 2, 4, 6))
    x = x.reshape(b_, c_ * 8, d_ // 2, h_ // 2, w_ // 2)
    x = jnp.squeeze(x, axis=2)
    x = jnp.transpose(x, (0, 2, 3, 1))
    return x.reshape(-1, K)


def _mega_kernel(s1_ref, s2_ref, zf2_ref, w_ref, z_ref,
                 idx_ref, out_ref, sums_ref):
    a2 = zf2_ref[...]
    s1 = s1_ref[...]
    col = lax.broadcasted_iota(jnp.int32, (NB, JC), 1).astype(jnp.float32)

    def scan_body(jc, carry):
        best, bidx = carry
        wc = w_ref[pl.ds(jc * JC, JC), :]
        mm2 = lax.dot_general(a2, wc, (((1,), (1,)), ((), ())),
                              preferred_element_type=jnp.float32)
        d = (s1 + s2_ref[:, pl.ds(jc * JC, JC)]) - mm2
        lmin = jnp.min(d, axis=1, keepdims=True)
        larg = jnp.min(jnp.where(d == lmin, col, jnp.float32(J)),
                       axis=1, keepdims=True)
        larg = larg + jnp.float32(jc * JC)
        take = lmin < best
        return (jnp.where(take, lmin, best), jnp.where(take, larg, bidx))

    best0 = jnp.full((NB, 1), jnp.inf, jnp.float32)
    bidx0 = jnp.zeros((NB, 1), jnp.float32)
    _, bidx_f = lax.fori_loop(0, NJC, scan_body, (best0, bidx0))
    idx_ref[0, :, :] = bidx_f.astype(jnp.int32)

    def gather_body(jc, zq_acc):
        wc = w_ref[pl.ds(jc * JC, JC), :]
        tgt = bidx_f - jnp.float32(jc * JC)
        oh = jnp.where(col == tgt, jnp.float32(1.0),
                       jnp.float32(0.0)).astype(jnp.bfloat16)
        return zq_acc + jnp.dot(oh, wc, preferred_element_type=jnp.float32)

    zq0 = jnp.zeros((NB, K), jnp.float32)
    zq = lax.fori_loop(0, NJC, gather_body, zq0)

    z = z_ref[...]
    diff = zq - z
    out_ref[...] = z + diff
    p = jnp.stack([
        jnp.sum(diff * diff),
        jnp.sum(zq),
        jnp.sum(z),
        jnp.sum(zq * z),
        jnp.sum(zq * zq),
        jnp.sum(z * z),
        jnp.float32(0.0), jnp.float32(0.0),
    ])
    sums_ref[0, 0, :] = p


def _mega(zf2b, wb, s1, s2, z_flat):
    idx, out, sums = pl.pallas_call(
        _mega_kernel,
        grid=(NI,),
        in_specs=[
            pl.BlockSpec((NB, 1), lambda i: (i, 0)),
            pl.BlockSpec((1, J), lambda i: (0, 0)),
            pl.BlockSpec((NB, K), lambda i: (i, 0)),
            pl.BlockSpec((J, K), lambda i: (0, 0)),
            pl.BlockSpec((NB, K), lambda i: (i, 0)),
        ],
        out_specs=[
            pl.BlockSpec((1, NB, 1), lambda i: (i, 0, 0)),
            pl.BlockSpec((NB, K), lambda i: (i, 0)),
            pl.BlockSpec((1, 1, 8), lambda i: (i, 0, 0)),
        ],
        out_shape=[
            jax.ShapeDtypeStruct((NI, NB, 1), jnp.int32),
            jax.ShapeDtypeStruct((N, K), jnp.float32),
            jax.ShapeDtypeStruct((NI, 1, 8), jnp.float32),
        ],
        compiler_params=pltpu.CompilerParams(
            dimension_semantics=("parallel",)),
    )(s1, s2, zf2b, wb, z_flat)
    return idx.reshape(N), out, jnp.sum(sums.reshape(NI, 8), axis=0)


_RPW = 128


def _sc_colsum(w):
    info = plsc.get_sparse_core_info()
    nw = info.num_cores * info.num_subcores
    rows_per_w = J // nw
    nslab = rows_per_w // _RPW
    ngrp = K // 16
    mesh = plsc.VectorSubcoreMesh(core_axis_name="c", subcore_axis_name="s")

    @functools.partial(
        pl.kernel, mesh=mesh,
        out_type=jax.ShapeDtypeStruct((nw, K), jnp.float32),
        scratch_types=[
            pltpu.VMEM((_RPW, K), jnp.float32),
            pltpu.VMEM((K,), jnp.float32),
        ],
    )
    def colsum_kernel(w_hbm, out_hbm, slab_v, acc_v):
        wid = lax.axis_index("s") * info.num_cores + lax.axis_index("c")
        zero = jnp.zeros((16,), jnp.float32)
        for g in range(ngrp):
            acc_v[pl.ds(g * 16, 16)] = zero
        for s in range(nslab):
            base = wid * rows_per_w + s * _RPW
            pltpu.sync_copy(w_hbm.at[pl.ds(base, _RPW)], slab_v)

            def row_body(r, carry):
                for g in range(ngrp):
                    x = jnp.abs(slab_v[r, pl.ds(g * 16, 16)])
                    plsc.addupdate(acc_v.at[pl.ds(g * 16, 16)], x)
                return carry

            lax.fori_loop(0, _RPW, row_body, jnp.int32(0))
        pltpu.sync_copy(acc_v, out_hbm.at[wid])

    return colsum_kernel(w)


def kernel(z, embedding_weight):
    w = embedding_weight
    zf = _unfold(z)
    s1 = jnp.sum(zf ** 2, axis=1, keepdims=True)
    s2 = jnp.sum(w ** 2, axis=1).reshape(1, J)
    zf2b = (zf * 2).astype(jnp.bfloat16)
    wb = w.astype(jnp.bfloat16)

    colsums = _sc_colsum(w)
    idx, out_flat, sums = _mega(zf2b, wb, s1, s2, z.reshape(N, K))

    reg = WEIGHT_DECAY * jnp.max(jnp.sum(colsums, axis=0))

    n_tot = jnp.float32(N * K)
    s_d2, s_q, s_z, s_qz, s_q2, s_z2 = [sums[k] for k in range(6)]
    sxy = s_qz - s_q * s_z / n_tot
    sxx = s_q2 - s_q * s_q / n_tot
    syy = s_z2 - s_z * s_z / n_tot
    cost = sxy / (jnp.sqrt(sxx) * jnp.sqrt(syy))
    pearson = 0.5 + 0.5 * cost
    m = s_d2 / n_tot
    loss = BETA * m + m + pearson + reg

    out = jnp.transpose(out_flat.reshape(z.shape), (0, 3, 1, 2))
    return out, loss, idx


# speedup vs baseline: 1.0299x; 1.0065x over previous
import functools

import jax
import jax.numpy as jnp
from jax import lax
from jax.experimental import pallas as pl
from jax.experimental.pallas import tpu as pltpu
from jax.experimental.pallas import tpu_sc as plsc

N = 4096
K = 512
J = 8192
NB = 512
JC = 2048
NI = N // NB
NJC = J // JC
BETA = 0.25
WEIGHT_DECAY = 0.01


def _unfold(z):
    x = z[:, :, None, :, :]
    b, c = x.shape[0], x.shape[1]
    x = jax.image.resize(x, (b, c, 2, 2, 2), method='trilinear')
    b_, c_, d_, h_, w_ = x.shape
    x = x.reshape(b_, c_, d_ // 2, 2, h_ // 2, 2, w_ // 2, 2)
    x = jnp.transpose(x, (0, 1, 3, 5, 7, 2, 4, 6))
    x = x.reshape(b_, c_ * 8, d_ // 2, h_ // 2, w_ // 2)
    x = jnp.squeeze(x, axis=2)
    x = jnp.transpose(x, (0, 2, 3, 1))
    return x.reshape(-1, K)


_WSL = J // NI


def _mega_kernel(s1_ref, s2_ref, zf2_ref, w_ref, z_ref,
                 idx_ref, out_ref, sums_ref, colsum_ref):
    a2 = zf2_ref[...]
    s1 = s1_ref[...]
    col = lax.broadcasted_iota(jnp.int32, (NB, JC), 1).astype(jnp.float32)

    def scan_body(jc, carry):
        best, bidx = carry
        wc = w_ref[pl.ds(jc * JC, JC), :]
        mm2 = lax.dot_general(a2, wc, (((1,), (1,)), ((), ())),
                              preferred_element_type=jnp.float32)
        d = (s1 + s2_ref[:, pl.ds(jc * JC, JC)]) - mm2
        lmin = jnp.min(d, axis=1, keepdims=True)
        larg = jnp.min(jnp.where(d == lmin, col, jnp.float32(J)),
                       axis=1, keepdims=True)
        larg = larg + jnp.float32(jc * JC)
        take = lmin < best
        return (jnp.where(take, lmin, best), jnp.where(take, larg, bidx))

    best0 = jnp.full((NB, 1), jnp.inf, jnp.float32)
    bidx0 = jnp.zeros((NB, 1), jnp.float32)
    _, bidx_f = lax.fori_loop(0, NJC, scan_body, (best0, bidx0))
    idx_ref[0, :, :] = bidx_f.astype(jnp.int32)

    def gather_body(jc, zq_acc):
        wc = w_ref[pl.ds(jc * JC, JC), :]
        tgt = bidx_f - jnp.float32(jc * JC)
        oh = jnp.where(col == tgt, jnp.float32(1.0),
                       jnp.float32(0.0)).astype(jnp.bfloat16)
        return zq_acc + jnp.dot(oh, wc, preferred_element_type=jnp.float32)

    zq0 = jnp.zeros((NB, K), jnp.float32)
    zq = lax.fori_loop(0, NJC, gather_body, zq0)

    z = z_ref[...]
    diff = zq - z
    out_ref[...] = z + diff
    p = jnp.stack([
        jnp.sum(diff * diff),
        jnp.sum(zq),
        jnp.sum(z),
        jnp.sum(zq * z),
        jnp.sum(zq * zq),
        jnp.sum(z * z),
        jnp.float32(0.0), jnp.float32(0.0),
    ])
    sums_ref[0, 0, :] = p

    i = pl.program_id(0)
    wsl = w_ref[pl.ds(i * _WSL, _WSL), :].astype(jnp.float32)
    colsum_ref[0, :, :] = jnp.sum(jnp.abs(wsl), axis=0, keepdims=True)


def _mega(zf2b, wb, s1, s2, z_flat):
    idx, out, sums, colsums = pl.pallas_call(
        _mega_kernel,
        grid=(NI,),
        in_specs=[
            pl.BlockSpec((NB, 1), lambda i: (i, 0)),
            pl.BlockSpec((1, J), lambda i: (0, 0)),
            pl.BlockSpec((NB, K), lambda i: (i, 0)),
            pl.BlockSpec((J, K), lambda i: (0, 0)),
            pl.BlockSpec((NB, K), lambda i: (i, 0)),
        ],
        out_specs=[
            pl.BlockSpec((1, NB, 1), lambda i: (i, 0, 0)),
            pl.BlockSpec((NB, K), lambda i: (i, 0)),
            pl.BlockSpec((1, 1, 8), lambda i: (i, 0, 0)),
            pl.BlockSpec((1, 1, K), lambda i: (i, 0, 0)),
        ],
        out_shape=[
            jax.ShapeDtypeStruct((NI, NB, 1), jnp.int32),
            jax.ShapeDtypeStruct((N, K), jnp.float32),
            jax.ShapeDtypeStruct((NI, 1, 8), jnp.float32),
            jax.ShapeDtypeStruct((NI, 1, K), jnp.float32),
        ],
        compiler_params=pltpu.CompilerParams(
            dimension_semantics=("parallel",)),
    )(s1, s2, zf2b, wb, z_flat)
    reg = WEIGHT_DECAY * jnp.max(jnp.sum(colsums.reshape(NI, K), axis=0))
    return idx.reshape(N), out, jnp.sum(sums.reshape(NI, 8), axis=0), reg


_RPW = 128


def _sc_colsum(w):
    info = plsc.get_sparse_core_info()
    nw = info.num_cores * info.num_subcores
    rows_per_w = J // nw
    nslab = rows_per_w // _RPW
    ngrp = K // 16
    mesh = plsc.VectorSubcoreMesh(core_axis_name="c", subcore_axis_name="s")

    @functools.partial(
        pl.kernel, mesh=mesh,
        out_type=jax.ShapeDtypeStruct((nw, K), jnp.float32),
        scratch_types=[
            pltpu.VMEM((_RPW, K), jnp.float32),
            pltpu.VMEM((K,), jnp.float32),
        ],
    )
    def colsum_kernel(w_hbm, out_hbm, slab_v, acc_v):
        wid = lax.axis_index("s") * info.num_cores + lax.axis_index("c")
        zero = jnp.zeros((16,), jnp.float32)
        for g in range(ngrp):
            acc_v[pl.ds(g * 16, 16)] = zero
        for s in range(nslab):
            base = wid * rows_per_w + s * _RPW
            pltpu.sync_copy(w_hbm.at[pl.ds(base, _RPW)], slab_v)

            def row_body(r, carry):
                for g in range(ngrp):
                    x = jnp.abs(slab_v[r, pl.ds(g * 16, 16)])
                    plsc.addupdate(acc_v.at[pl.ds(g * 16, 16)], x)
                return carry

            lax.fori_loop(0, _RPW, row_body, jnp.int32(0))
        pltpu.sync_copy(acc_v, out_hbm.at[wid])

    return colsum_kernel(w)


def kernel(z, embedding_weight):
    w = embedding_weight
    zf = _unfold(z)
    s1 = jnp.sum(zf ** 2, axis=1, keepdims=True)
    s2 = jnp.sum(w ** 2, axis=1).reshape(1, J)
    zf2b = (zf * 2).astype(jnp.bfloat16)
    wb = w.astype(jnp.bfloat16)

    idx, out_flat, sums, reg = _mega(zf2b, wb, s1, s2, z.reshape(N, K))

    n_tot = jnp.float32(N * K)
    s_d2, s_q, s_z, s_qz, s_q2, s_z2 = [sums[k] for k in range(6)]
    sxy = s_qz - s_q * s_z / n_tot
    sxx = s_q2 - s_q * s_q / n_tot
    syy = s_z2 - s_z * s_z / n_tot
    cost = sxy / (jnp.sqrt(sxx) * jnp.sqrt(syy))
    pearson = 0.5 + 0.5 * cost
    m = s_d2 / n_tot
    loss = BETA * m + m + pearson + reg

    out = jnp.transpose(out_flat.reshape(z.shape), (0, 3, 1, 2))
    return out, loss, idx


# speedup vs baseline: 1.1759x; 1.1418x over previous
import functools

import jax
import jax.numpy as jnp
from jax import lax
from jax.experimental import pallas as pl
from jax.experimental.pallas import tpu as pltpu
from jax.experimental.pallas import tpu_sc as plsc

N = 4096
K = 512
J = 8192
NB = 512
JC = 2048
NI = N // NB
NJC = J // JC
BETA = 0.25
WEIGHT_DECAY = 0.01


def _unfold(z):
    x = z[:, :, None, :, :]
    b, c = x.shape[0], x.shape[1]
    x = jax.image.resize(x, (b, c, 2, 2, 2), method='trilinear')
    b_, c_, d_, h_, w_ = x.shape
    x = x.reshape(b_, c_, d_ // 2, 2, h_ // 2, 2, w_ // 2, 2)
    x = jnp.transpose(x, (0, 1, 3, 5, 7, 2, 4, 6))
    x = x.reshape(b_, c_ * 8, d_ // 2, h_ // 2, w_ // 2)
    x = jnp.squeeze(x, axis=2)
    x = jnp.transpose(x, (0, 2, 3, 1))
    return x.reshape(-1, K)


_WSL = J // NI


def _dist_kernel(s1_ref, s2_ref, zf2_ref, w_ref, idx_ref, colsum_ref):
    a2 = zf2_ref[...]
    s1 = s1_ref[...]
    col = lax.broadcasted_iota(jnp.int32, (NB, JC), 1).astype(jnp.float32)

    def scan_body(jc, carry):
        best, bidx = carry
        wc = w_ref[pl.ds(jc * JC, JC), :]
        mm2 = lax.dot_general(a2, wc, (((1,), (1,)), ((), ())),
                              preferred_element_type=jnp.float32)
        d = (s1 + s2_ref[:, pl.ds(jc * JC, JC)]) - mm2
        lmin = jnp.min(d, axis=1, keepdims=True)
        larg = jnp.min(jnp.where(d == lmin, col, jnp.float32(J)),
                       axis=1, keepdims=True)
        larg = larg + jnp.float32(jc * JC)
        take = lmin < best
        return (jnp.where(take, lmin, best), jnp.where(take, larg, bidx))

    best0 = jnp.full((NB, 1), jnp.inf, jnp.float32)
    bidx0 = jnp.zeros((NB, 1), jnp.float32)
    _, bidx_f = lax.fori_loop(0, NJC, scan_body, (best0, bidx0))
    idx_ref[0, :, :] = bidx_f.astype(jnp.int32)

    i = pl.program_id(0)
    wsl = w_ref[pl.ds(i * _WSL, _WSL), :].astype(jnp.float32)
    colsum_ref[0, :, :] = jnp.sum(jnp.abs(wsl), axis=0, keepdims=True)


def _dist_argmin(zf2b, wb, s1, s2):
    idx, colsums = pl.pallas_call(
        _dist_kernel,
        grid=(NI,),
        in_specs=[
            pl.BlockSpec((NB, 1), lambda i: (i, 0)),
            pl.BlockSpec((1, J), lambda i: (0, 0)),
            pl.BlockSpec((NB, K), lambda i: (i, 0)),
            pl.BlockSpec((J, K), lambda i: (0, 0)),
        ],
        out_specs=[
            pl.BlockSpec((1, NB, 1), lambda i: (i, 0, 0)),
            pl.BlockSpec((1, 1, K), lambda i: (i, 0, 0)),
        ],
        out_shape=[
            jax.ShapeDtypeStruct((NI, NB, 1), jnp.int32),
            jax.ShapeDtypeStruct((NI, 1, K), jnp.float32),
        ],
        compiler_params=pltpu.CompilerParams(
            dimension_semantics=("parallel",)),
    )(s1, s2, zf2b, wb)
    reg = WEIGHT_DECAY * jnp.max(jnp.sum(colsums.reshape(NI, K), axis=0))
    return idx.reshape(N), reg


def _sc_gather(w, idx):
    info = plsc.get_sparse_core_info()
    nw = info.num_cores * info.num_subcores
    b_per_w = N // nw
    mesh = plsc.VectorSubcoreMesh(core_axis_name="c", subcore_axis_name="s")

    @functools.partial(
        pl.kernel, mesh=mesh,
        out_type=jax.ShapeDtypeStruct((N, K), jnp.float32),
        scratch_types=[
            pltpu.VMEM((b_per_w,), jnp.int32),
            pltpu.VMEM((b_per_w, K), jnp.float32),
            pltpu.SemaphoreType.DMA,
        ],
    )
    def gather_kernel(table_hbm, idx_hbm, out_hbm, idx_v, rows_v, sem):
        wid = lax.axis_index("s") * info.num_cores + lax.axis_index("c")
        base = wid * b_per_w
        pltpu.sync_copy(idx_hbm.at[pl.ds(base, b_per_w)], idx_v)
        pltpu.async_copy(table_hbm.at[idx_v], rows_v, sem).wait()
        pltpu.sync_copy(rows_v, out_hbm.at[pl.ds(base, b_per_w)])

    return gather_kernel(w, idx)


_RB = 512
_RG = N // _RB


def _loss_out_kernel(z_ref, zq_ref, out_ref, sums_ref):
    z = z_ref[...]
    zq = zq_ref[...]
    diff = zq - z
    out_ref[...] = z + diff

    p = jnp.stack([
        jnp.sum(diff * diff),
        jnp.sum(zq),
        jnp.sum(z),
        jnp.sum(zq * z),
        jnp.sum(zq * zq),
        jnp.sum(z * z),
        jnp.float32(0.0), jnp.float32(0.0),
    ])
    sums_ref[0, 0, :] = p


def _loss_out(z_flat, zq):
    out, sums = pl.pallas_call(
        _loss_out_kernel,
        grid=(_RG,),
        in_specs=[
            pl.BlockSpec((_RB, K), lambda g: (g, 0)),
            pl.BlockSpec((_RB, K), lambda g: (g, 0)),
        ],
        out_specs=[
            pl.BlockSpec((_RB, K), lambda g: (g, 0)),
            pl.BlockSpec((1, 1, 8), lambda g: (g, 0, 0)),
        ],
        out_shape=[
            jax.ShapeDtypeStruct((N, K), jnp.float32),
            jax.ShapeDtypeStruct((_RG, 1, 8), jnp.float32),
        ],
        compiler_params=pltpu.CompilerParams(
            dimension_semantics=("parallel",)),
    )(z_flat, zq)
    return out, jnp.sum(sums.reshape(_RG, 8), axis=0)


def kernel(z, embedding_weight):
    w = embedding_weight
    zf = _unfold(z)
    s1 = jnp.sum(zf ** 2, axis=1, keepdims=True)
    s2 = jnp.sum(w ** 2, axis=1).reshape(1, J)
    zf2b = (zf * 2).astype(jnp.bfloat16)
    wb = w.astype(jnp.bfloat16)

    idx, reg = _dist_argmin(zf2b, wb, s1, s2)
    zq = _sc_gather(w, idx)

    out_flat, sums = _loss_out(z.reshape(N, K), zq)

    n_tot = jnp.float32(N * K)
    s_d2, s_q, s_z, s_qz, s_q2, s_z2 = [sums[k] for k in range(6)]
    sxy = s_qz - s_q * s_z / n_tot
    sxx = s_q2 - s_q * s_q / n_tot
    syy = s_z2 - s_z * s_z / n_tot
    cost = sxy / (jnp.sqrt(sxx) * jnp.sqrt(syy))
    pearson = 0.5 + 0.5 * cost
    m = s_d2 / n_tot
    loss = BETA * m + m + pearson + reg

    out = jnp.transpose(out_flat.reshape(z.shape), (0, 3, 1, 2))
    return out, loss, idx
